# Initial kernel scaffold; baseline (speedup 1.0000x reference)
import jax
import jax.numpy as jnp
from jax.experimental import pallas as pl


def kernel(x, edge_index, Wl1, bl1, Wr1, Wl2, bl2, Wr2, Wd1, bd1, Wd2, bd2):
    raise NotImplementedError("write your pallas kernel here")


# speedup vs baseline: 2.2801x; 2.2801x over previous
import functools

import jax
import jax.numpy as jnp
from jax import lax
from jax.experimental import pallas as pl
from jax.experimental.pallas import tpu as pltpu
from jax.experimental.pallas import tpu_sc as plsc

_N = 10000
_E = 320000
_H = 128
_NC = 2
_NS = 16
_NW = _NC * _NS
_N2 = 10240
_RPS = _N2 // _NS
_CHUNKS = 2560
_E_PAD = _CHUNKS * 128
_CPG = 2
_SEG_GPW = _CHUNKS // _CPG // _NW
_DCPG = 2
_DEC_GPW = _CHUNKS // _DCPG // _NW

def _sc_mesh():
    return plsc.VectorSubcoreMesh(core_axis_name="c", subcore_axis_name="s",
                                  num_cores=_NC, num_subcores=_NS)


@functools.cache
def _build_sc_segsum():
    @functools.partial(
        pl.kernel,
        out_type=jax.ShapeDtypeStruct((_NC, _N2, _H), jnp.float32),
        mesh=_sc_mesh(),
        scratch_types=[
            pltpu.VMEM((_CPG, 128), jnp.int32),
            pltpu.VMEM((_CPG, 128), jnp.int32),
            pltpu.VMEM((_CPG * 128, _H), jnp.float32),
            pltpu.VMEM_SHARED((_N2, _H), jnp.float32),
        ],
    )
    def k(h_hbm, s_hbm, d_hbm, zrow_hbm, agg_hbm, sidx, didx, rows, acc):
        c = lax.axis_index("c")
        s = lax.axis_index("s")
        wid = s * _NC + c
        base = s * _RPS
        pltpu.sync_copy(zrow_hbm, acc.at[pl.ds(base, _RPS)])
        plsc.subcore_barrier()

        @pl.loop(0, _SEG_GPW)
        def _(gi):
            g = gi * _NW + wid
            pltpu.sync_copy(s_hbm.at[pl.ds(g * _CPG, _CPG)], sidx)
            pltpu.sync_copy(d_hbm.at[pl.ds(g * _CPG, _CPG)], didx)
            for j in range(_CPG):
                pltpu.sync_copy(h_hbm.at[sidx.at[j]],
                                rows.at[pl.ds(j * 128, 128)])
            for j in range(_CPG):
                pltpu.sync_copy(rows.at[pl.ds(j * 128, 128)],
                                acc.at[didx.at[j]], add=True)

        plsc.subcore_barrier()
        pltpu.sync_copy(acc.at[pl.ds(base, _RPS)],
                        agg_hbm.at[c].at[pl.ds(base, _RPS)])

    return k


def _sc_segsum(h, s2d, d2d, zrow):
    return _build_sc_segsum()(h, s2d, d2d, zrow)


@functools.cache
def _build_sc_counts():
    @functools.partial(
        pl.kernel,
        out_type=jax.ShapeDtypeStruct((_NC, _N2, _H), jnp.float32),
        mesh=_sc_mesh(),
        scratch_types=[
            pltpu.VMEM((1, 128), jnp.int32),
            pltpu.VMEM((128, _H), jnp.float32),
            pltpu.VMEM_SHARED((_N2, _H), jnp.float32),
        ],
    )
    def k(d_hbm, zrow_hbm, ones_hbm, cnt_hbm, didx, ones_v, cacc):
        c = lax.axis_index("c")
        s = lax.axis_index("s")
        wid = s * _NC + c
        base = s * _RPS
        pltpu.sync_copy(zrow_hbm, cacc.at[pl.ds(base, _RPS)])
        pltpu.sync_copy(ones_hbm, ones_v)
        plsc.subcore_barrier()

        @pl.loop(0, _CHUNKS // _NW)
        def _(gi):
            g = gi * _NW + wid
            pltpu.sync_copy(d_hbm.at[pl.ds(g, 1)], didx)
            pltpu.sync_copy(ones_v, cacc.at[didx.at[0]], add=True)

        plsc.subcore_barrier()
        pltpu.sync_copy(cacc.at[pl.ds(base, _RPS)],
                        cnt_hbm.at[c].at[pl.ds(base, _RPS)])

    return k


def _sc_counts(d2d, zrow, ones128):
    return _build_sc_counts()(d2d, zrow, ones128)


@functools.cache
def _build_sc_edge_gather():
    @functools.partial(
        pl.kernel,
        out_type=(
            jax.ShapeDtypeStruct((_E_PAD, _H), jnp.float32),
            jax.ShapeDtypeStruct((_E_PAD, _H), jnp.float32),
        ),
        mesh=_sc_mesh(),
        scratch_types=[
            pltpu.VMEM((_DCPG, 128), jnp.int32),
            pltpu.VMEM((_DCPG, 128), jnp.int32),
            pltpu.VMEM((_DCPG * 128, _H), jnp.float32),
            pltpu.VMEM((_DCPG * 128, _H), jnp.float32),
        ],
    )
    def k(a_hbm, b_hbm, s_hbm, d_hbm, ga_hbm, gb_hbm,
          sidx, didx, rows_a, rows_b):
        c = lax.axis_index("c")
        s = lax.axis_index("s")
        wid = s * _NC + c

        @pl.loop(0, _DEC_GPW)
        def _(gi):
            g = gi * _NW + wid
            pltpu.sync_copy(s_hbm.at[pl.ds(g * _DCPG, _DCPG)], sidx)
            pltpu.sync_copy(d_hbm.at[pl.ds(g * _DCPG, _DCPG)], didx)
            for j in range(_DCPG):
                pltpu.sync_copy(a_hbm.at[sidx.at[j]],
                                rows_a.at[pl.ds(j * 128, 128)])
                pltpu.sync_copy(b_hbm.at[didx.at[j]],
                                rows_b.at[pl.ds(j * 128, 128)])
            pltpu.sync_copy(rows_a,
                            ga_hbm.at[pl.ds(g * _DCPG * 128, _DCPG * 128)])
            pltpu.sync_copy(rows_b,
                            gb_hbm.at[pl.ds(g * _DCPG * 128, _DCPG * 128)])

    return k


def _sc_edge_gather(a, b, s2d, d2d):
    return _build_sc_edge_gather()(a, b, s2d, d2d)


def _tc_sage_body(relu, a0, a1, c0, c1, h, wlt, bl, wrt, o):
    cnt = jnp.maximum(c0[...] + c1[...], 1.0)
    agg = (a0[...] + a1[...]) / cnt
    v = (jnp.dot(agg, wlt[...], preferred_element_type=jnp.float32) + bl[...]
         + jnp.dot(h[...], wrt[...], preferred_element_type=jnp.float32))
    o[...] = jnp.maximum(v, 0.0) if relu else v


def _tc_sage(relu, a0, a1, c0, c1, h, wlt, bl, wrt):
    return pl.pallas_call(
        functools.partial(_tc_sage_body, relu),
        out_shape=jax.ShapeDtypeStruct((_N, _H), jnp.float32),
    )(a0, a1, c0, c1, h, wlt, bl, wrt)


def _tc_decode_prep_body(a0, a1, c0, c1, z1, wlt, bl, wrt, wdlt, wdrt, bd1,
                         oa, ob):
    cnt = jnp.maximum(c0[...] + c1[...], 1.0)
    agg = (a0[...] + a1[...]) / cnt
    z2 = (jnp.dot(agg, wlt[...], preferred_element_type=jnp.float32) + bl[...]
          + jnp.dot(z1[...], wrt[...], preferred_element_type=jnp.float32))
    oa[...] = jnp.dot(z2, wdlt[...], preferred_element_type=jnp.float32) + bd1[...]
    ob[...] = jnp.dot(z2, wdrt[...], preferred_element_type=jnp.float32)


def _tc_decode_prep(a0, a1, c0, c1, z1, wlt, bl, wrt, wdlt, wdrt, bd1):
    return pl.pallas_call(
        _tc_decode_prep_body,
        out_shape=(jax.ShapeDtypeStruct((_N, _H), jnp.float32),
                   jax.ShapeDtypeStruct((_N, _H), jnp.float32)),
    )(a0, a1, c0, c1, z1, wlt, bl, wrt, wdlt, wdrt, bd1)


def _tc_logits_body(ga, gb, w2, b2, o):
    r = jnp.maximum(ga[...] + gb[...], 0.0)
    o[...] = jnp.sum(r * w2[...], axis=1, keepdims=True) + b2[...]


_LBLK = 4096


def _tc_logits(ga, gb, w2, b2):
    nblk = _E_PAD // _LBLK
    return pl.pallas_call(
        _tc_logits_body,
        grid=(nblk,),
        in_specs=[
            pl.BlockSpec((_LBLK, _H), lambda i: (i, 0)),
            pl.BlockSpec((_LBLK, _H), lambda i: (i, 0)),
            pl.BlockSpec((1, _H), lambda i: (0, 0)),
            pl.BlockSpec((1, 1), lambda i: (0, 0)),
        ],
        out_specs=pl.BlockSpec((_LBLK, 1), lambda i: (i, 0)),
        out_shape=jax.ShapeDtypeStruct((_E_PAD, 1), jnp.float32),
    )(ga, gb, w2, b2)


def kernel(x, edge_index, Wl1, bl1, Wr1, Wl2, bl2, Wr2, Wd1, bd1, Wd2, bd2):
    src = edge_index[0]
    dst = edge_index[1]
    npad = _E_PAD - _E
    src2d = jnp.concatenate([src, jnp.zeros((npad,), jnp.int32)]).reshape(_CHUNKS, 128)
    dst2d_seg = jnp.concatenate(
        [dst, jnp.full((npad,), _N2 - 1, jnp.int32)]).reshape(_CHUNKS, 128)
    dst2d_dec = jnp.concatenate(
        [dst, jnp.zeros((npad,), jnp.int32)]).reshape(_CHUNKS, 128)

    zrow = jnp.zeros((_RPS, _H), jnp.float32)
    ones128 = jnp.ones((128, _H), jnp.float32)

    wl1t = Wl1.T
    wr1t = Wr1.T
    wl2t = Wl2.T
    wr2t = Wr2.T
    wdlt = Wd1[:, :_H].T
    wdrt = Wd1[:, _H:].T
    bl1r = bl1.reshape(1, _H)
    bl2r = bl2.reshape(1, _H)
    bd1r = bd1.reshape(1, _H)
    w2r = Wd2.reshape(1, _H)
    b2r = bd2.reshape(1, 1)

    cnt = _sc_counts(dst2d_seg, zrow, ones128)
    zrow1 = zrow + cnt[0, 0, 0] * 0.0
    agg1 = _sc_segsum(x, src2d, dst2d_seg, zrow1)
    c0 = cnt[0, :_N, 0:1]
    c1 = cnt[1, :_N, 0:1]
    z1 = _tc_sage(True, agg1[0, :_N], agg1[1, :_N], c0, c1, x, wl1t, bl1r, wr1t)

    agg2 = _sc_segsum(z1, src2d, dst2d_seg, zrow)
    A, B = _tc_decode_prep(agg2[0, :_N], agg2[1, :_N], c0, c1, z1,
                           wl2t, bl2r, wr2t, wdlt, wdrt, bd1r)

    ga, gb = _sc_edge_gather(A, B, src2d, dst2d_dec)
    logits = _tc_logits(ga, gb, w2r, b2r)
    return logits[:_E, 0]


# speedup vs baseline: 2.5054x; 1.0988x over previous
import functools

import jax
import jax.numpy as jnp
from jax import lax
from jax.experimental import pallas as pl
from jax.experimental.pallas import tpu as pltpu
from jax.experimental.pallas import tpu_sc as plsc

_N = 10000
_E = 320000
_H = 128
_NC = 2
_NS = 16
_NW = _NC * _NS
_N2 = 10240
_RPS = _N2 // _NS
_CHUNKS = 2560
_E_PAD = _CHUNKS * 128
_SEG_GPW = _CHUNKS // _NW
_DEC_GPW = _CHUNKS // _NW

def _sc_mesh():
    return plsc.VectorSubcoreMesh(core_axis_name="c", subcore_axis_name="s",
                                  num_cores=_NC, num_subcores=_NS)


@functools.cache
def _build_sc_segsum():
    @functools.partial(
        pl.kernel,
        out_type=jax.ShapeDtypeStruct((_NC, _N2, _H), jnp.float32),
        mesh=_sc_mesh(),
        scratch_types=[
            pltpu.VMEM((2, 128), jnp.int32),
            pltpu.VMEM((2, 128), jnp.int32),
            pltpu.VMEM((128, _H), jnp.float32),
            pltpu.VMEM((128, _H), jnp.float32),
            pltpu.VMEM_SHARED((_N2, _H), jnp.float32),
            pltpu.SemaphoreType.DMA,
            pltpu.SemaphoreType.DMA,
            pltpu.SemaphoreType.DMA,
            pltpu.SemaphoreType.DMA,
        ],
    )
    def k(h_hbm, ci_hbm, zrow_hbm, agg_hbm,
          cidx0, cidx1, rows0, rows1, acc, sg0, sg1, ss0, ss1):
        c = lax.axis_index("c")
        s = lax.axis_index("s")
        wid = s * _NC + c
        base = s * _RPS
        cidx = (cidx0, cidx1)
        rows = (rows0, rows1)
        sg = (sg0, sg1)
        ss = (ss0, ss1)
        pltpu.sync_copy(zrow_hbm, acc.at[pl.ds(base, _RPS)])
        plsc.subcore_barrier()

        for b in range(2):
            g = b * _NW + wid
            pltpu.sync_copy(ci_hbm.at[pl.ds(g * 2, 2)], cidx[b])
            pltpu.async_copy(h_hbm.at[cidx[b].at[0]], rows[b], sg[b])

        @pl.loop(0, _SEG_GPW // 2)
        def _(i):
            for b in range(2):
                gi = i * 2 + b
                pltpu.make_async_copy(h_hbm.at[cidx[b].at[0]], rows[b],
                                      sg[b]).wait()
                pltpu.async_copy(rows[b], acc.at[cidx[b].at[1]], ss[b],
                                 add=True)

                @pl.when(gi + 2 < _SEG_GPW)
                def _():
                    pltpu.make_async_copy(rows[b], acc.at[cidx[b].at[1]],
                                          ss[b]).wait()
                    g2 = (gi + 2) * _NW + wid
                    pltpu.sync_copy(ci_hbm.at[pl.ds(g2 * 2, 2)], cidx[b])
                    pltpu.async_copy(h_hbm.at[cidx[b].at[0]], rows[b], sg[b])

        for b in range(2):
            pltpu.make_async_copy(rows[b], acc.at[cidx[b].at[1]], ss[b]).wait()
        plsc.subcore_barrier()
        pltpu.sync_copy(acc.at[pl.ds(base, _RPS)],
                        agg_hbm.at[c].at[pl.ds(base, _RPS)])

    return k


def _sc_segsum(h, ci2d, zrow):
    return _build_sc_segsum()(h, ci2d, zrow)


@functools.cache
def _build_sc_counts():
    @functools.partial(
        pl.kernel,
        out_type=jax.ShapeDtypeStruct((_NC, _N2, _H), jnp.float32),
        mesh=_sc_mesh(),
        scratch_types=[
            pltpu.VMEM((1, 128), jnp.int32),
            pltpu.VMEM((128, _H), jnp.float32),
            pltpu.VMEM_SHARED((_N2, _H), jnp.float32),
        ],
    )
    def k(d_hbm, zrow_hbm, ones_hbm, cnt_hbm, didx, ones_v, cacc):
        c = lax.axis_index("c")
        s = lax.axis_index("s")
        wid = s * _NC + c
        base = s * _RPS
        pltpu.sync_copy(zrow_hbm, cacc.at[pl.ds(base, _RPS)])
        pltpu.sync_copy(ones_hbm, ones_v)
        plsc.subcore_barrier()

        @pl.loop(0, _CHUNKS // _NW)
        def _(gi):
            g = gi * _NW + wid
            pltpu.sync_copy(d_hbm.at[pl.ds(g, 1)], didx)
            pltpu.sync_copy(ones_v, cacc.at[didx.at[0]], add=True)

        plsc.subcore_barrier()
        pltpu.sync_copy(cacc.at[pl.ds(base, _RPS)],
                        cnt_hbm.at[c].at[pl.ds(base, _RPS)])

    return k


def _sc_counts(d2d, zrow, ones128):
    return _build_sc_counts()(d2d, zrow, ones128)


@functools.cache
def _build_sc_edge_gather():
    @functools.partial(
        pl.kernel,
        out_type=jax.ShapeDtypeStruct((_CHUNKS * 256, _H), jnp.float32),
        mesh=_sc_mesh(),
        scratch_types=[
            pltpu.VMEM((2, 128), jnp.int32),
            pltpu.VMEM((2, 128), jnp.int32),
            pltpu.VMEM((256, _H), jnp.float32),
            pltpu.VMEM((256, _H), jnp.float32),
            pltpu.SemaphoreType.DMA,
            pltpu.SemaphoreType.DMA,
            pltpu.SemaphoreType.DMA,
            pltpu.SemaphoreType.DMA,
        ],
    )
    def k(a_hbm, b_hbm, ci_hbm, gab_hbm,
          cidx0, cidx1, rows0, rows1, sg0, sg1, sw0, sw1):
        c = lax.axis_index("c")
        s = lax.axis_index("s")
        wid = s * _NC + c
        cidx = (cidx0, cidx1)
        rows = (rows0, rows1)
        sg = (sg0, sg1)
        sw = (sw0, sw1)

        def start_gathers(b):
            pltpu.async_copy(a_hbm.at[cidx[b].at[0]],
                             rows[b].at[pl.ds(0, 128)], sg[b])
            pltpu.async_copy(b_hbm.at[cidx[b].at[1]],
                             rows[b].at[pl.ds(128, 128)], sg[b])

        def wait_gathers(b):
            pltpu.make_async_copy(a_hbm.at[cidx[b].at[0]],
                                  rows[b].at[pl.ds(0, 128)], sg[b]).wait()
            pltpu.make_async_copy(b_hbm.at[cidx[b].at[1]],
                                  rows[b].at[pl.ds(128, 128)], sg[b]).wait()

        for b in range(2):
            g = b * _NW + wid
            pltpu.sync_copy(ci_hbm.at[pl.ds(g * 2, 2)], cidx[b])
            start_gathers(b)

        @pl.loop(0, _DEC_GPW // 2)
        def _(i):
            for b in range(2):
                gi = i * 2 + b
                g = gi * _NW + wid
                wait_gathers(b)
                pltpu.async_copy(rows[b], gab_hbm.at[pl.ds(g * 256, 256)],
                                 sw[b])

                @pl.when(gi + 2 < _DEC_GPW)
                def _():
                    pltpu.make_async_copy(rows[b],
                                          gab_hbm.at[pl.ds(g * 256, 256)],
                                          sw[b]).wait()
                    g2 = (gi + 2) * _NW + wid
                    pltpu.sync_copy(ci_hbm.at[pl.ds(g2 * 2, 2)], cidx[b])
                    start_gathers(b)

        for b in range(2):
            pltpu.make_async_copy(rows[b], gab_hbm.at[pl.ds(0, 256)],
                                  sw[b]).wait()

    return k


def _sc_edge_gather(a, b, ci2d):
    return _build_sc_edge_gather()(a, b, ci2d)


def _tc_sage_body(relu, a0, a1, c0, c1, h, wlt, bl, wrt, o):
    cnt = jnp.maximum(c0[...] + c1[...], 1.0)
    agg = (a0[...] + a1[...]) / cnt
    v = (jnp.dot(agg, wlt[...], preferred_element_type=jnp.float32) + bl[...]
         + jnp.dot(h[...], wrt[...], preferred_element_type=jnp.float32))
    o[...] = jnp.maximum(v, 0.0) if relu else v


def _tc_sage(relu, a0, a1, c0, c1, h, wlt, bl, wrt):
    return pl.pallas_call(
        functools.partial(_tc_sage_body, relu),
        out_shape=jax.ShapeDtypeStruct((_N, _H), jnp.float32),
    )(a0, a1, c0, c1, h, wlt, bl, wrt)


def _tc_decode_prep_body(a0, a1, c0, c1, z1, wlt, bl, wrt, wdlt, wdrt, bd1,
                         oa, ob):
    cnt = jnp.maximum(c0[...] + c1[...], 1.0)
    agg = (a0[...] + a1[...]) / cnt
    z2 = (jnp.dot(agg, wlt[...], preferred_element_type=jnp.float32) + bl[...]
          + jnp.dot(z1[...], wrt[...], preferred_element_type=jnp.float32))
    oa[...] = jnp.dot(z2, wdlt[...], preferred_element_type=jnp.float32) + bd1[...]
    ob[...] = jnp.dot(z2, wdrt[...], preferred_element_type=jnp.float32)


def _tc_decode_prep(a0, a1, c0, c1, z1, wlt, bl, wrt, wdlt, wdrt, bd1):
    return pl.pallas_call(
        _tc_decode_prep_body,
        out_shape=(jax.ShapeDtypeStruct((_N, _H), jnp.float32),
                   jax.ShapeDtypeStruct((_N, _H), jnp.float32)),
    )(a0, a1, c0, c1, z1, wlt, bl, wrt, wdlt, wdrt, bd1)


_LCHUNKS = 8


def _tc_logits_body(gab, w2, b2, o):
    v = gab[...]
    ga = jnp.concatenate(
        [v[k * 256:k * 256 + 128] for k in range(_LCHUNKS)], axis=0)
    gb = jnp.concatenate(
        [v[k * 256 + 128:k * 256 + 256] for k in range(_LCHUNKS)], axis=0)
    r = jnp.maximum(ga + gb, 0.0)
    o[...] = jnp.sum(r * w2[...], axis=1, keepdims=True) + b2[...]


def _tc_logits(gab, w2, b2):
    nblk = _CHUNKS // _LCHUNKS
    return pl.pallas_call(
        _tc_logits_body,
        grid=(nblk,),
        in_specs=[
            pl.BlockSpec((_LCHUNKS * 256, _H), lambda i: (i, 0)),
            pl.BlockSpec((1, _H), lambda i: (0, 0)),
            pl.BlockSpec((1, 1), lambda i: (0, 0)),
        ],
        out_specs=pl.BlockSpec((_LCHUNKS * 128, 1), lambda i: (i, 0)),
        out_shape=jax.ShapeDtypeStruct((_E_PAD, 1), jnp.float32),
    )(gab, w2, b2)


def kernel(x, edge_index, Wl1, bl1, Wr1, Wl2, bl2, Wr2, Wd1, bd1, Wd2, bd2):
    src = edge_index[0]
    dst = edge_index[1]
    npad = _E_PAD - _E
    src2d = jnp.concatenate([src, jnp.zeros((npad,), jnp.int32)]).reshape(_CHUNKS, 128)
    dst2d_seg = jnp.concatenate(
        [dst, jnp.full((npad,), _N2 - 1, jnp.int32)]).reshape(_CHUNKS, 128)
    dst2d_dec = jnp.concatenate(
        [dst, jnp.zeros((npad,), jnp.int32)]).reshape(_CHUNKS, 128)
    ci_seg = jnp.stack([src2d, dst2d_seg], axis=1).reshape(_CHUNKS * 2, 128)
    ci_dec = jnp.stack([src2d, dst2d_dec], axis=1).reshape(_CHUNKS * 2, 128)

    zrow = jnp.zeros((_RPS, _H), jnp.float32)
    ones128 = jnp.ones((128, _H), jnp.float32)

    wl1t = Wl1.T
    wr1t = Wr1.T
    wl2t = Wl2.T
    wr2t = Wr2.T
    wdlt = Wd1[:, :_H].T
    wdrt = Wd1[:, _H:].T
    bl1r = bl1.reshape(1, _H)
    bl2r = bl2.reshape(1, _H)
    bd1r = bd1.reshape(1, _H)
    w2r = Wd2.reshape(1, _H)
    b2r = bd2.reshape(1, 1)

    cnt = _sc_counts(dst2d_seg, zrow, ones128)
    zrow1 = zrow + cnt[0, 0, 0] * 0.0
    agg1 = _sc_segsum(x, ci_seg, zrow1)
    c0 = cnt[0, :_N, 0:1]
    c1 = cnt[1, :_N, 0:1]
    z1 = _tc_sage(True, agg1[0, :_N], agg1[1, :_N], c0, c1, x, wl1t, bl1r, wr1t)

    agg2 = _sc_segsum(z1, ci_seg, zrow)
    A, B = _tc_decode_prep(agg2[0, :_N], agg2[1, :_N], c0, c1, z1,
                           wl2t, bl2r, wr2t, wdlt, wdrt, bd1r)

    gab = _sc_edge_gather(A, B, ci_dec)
    logits = _tc_logits(gab, w2r, b2r)
    return logits[:_E, 0]
